# baseline (device time: 13058 ns/iter reference)
import jax
import jax.numpy as jnp
from jax import lax
from jax.experimental import pallas as pl
from jax.experimental.pallas import tpu as pltpu

N_CHUNKS = 4


def kernel(A, B):
    m, k = A.shape
    _, n = B.shape
    nc = n // N_CHUNKS

    def body(a_hbm, b_hbm, out_hbm, a_f32, b_f32, a_bf, b_bf,
             a_rcv, b_rcv, acc, in_sems, out_sems, send_sems, recv_sems):
        my_x = lax.axis_index("x")
        my_y = lax.axis_index("y")
        partner = (1 - my_x, my_y)

        barrier_sem = pltpu.get_barrier_semaphore()
        pl.semaphore_signal(
            barrier_sem, inc=1,
            device_id=partner, device_id_type=pl.DeviceIdType.MESH,
        )

        cp_a = pltpu.make_async_copy(a_hbm, a_f32, in_sems.at[0])
        cp_b = pltpu.make_async_copy(b_hbm, b_f32, in_sems.at[1])
        cp_a.start()
        cp_b.start()
        cp_a.wait()
        cp_b.wait()

        a_bf[...] = a_f32[...].astype(jnp.bfloat16)
        for j in range(N_CHUNKS):
            b_bf[j] = b_f32[:, pl.ds(j * nc, nc)].astype(jnp.bfloat16)

        pl.semaphore_wait(barrier_sem, 1)

        rdma_a = pltpu.make_async_remote_copy(
            src_ref=a_bf, dst_ref=a_rcv,
            send_sem=send_sems.at[0], recv_sem=recv_sems.at[0],
            device_id=partner, device_id_type=pl.DeviceIdType.MESH,
        )
        rdma_a.start()
        rdma_bs = []
        for j in range(N_CHUNKS):
            r = pltpu.make_async_remote_copy(
                src_ref=b_bf.at[j], dst_ref=b_rcv.at[j],
                send_sem=send_sems.at[1 + j], recv_sem=recv_sems.at[1 + j],
                device_id=partner, device_id_type=pl.DeviceIdType.MESH,
            )
            r.start()
            rdma_bs.append(r)

        for j in range(N_CHUNKS):
            acc[j] = jnp.dot(
                a_bf[...], b_bf[j], preferred_element_type=jnp.float32
            )

        rdma_a.wait_recv()
        out_cps = []
        for j in range(N_CHUNKS):
            rdma_bs[j].wait_recv()
            acc[j] += jnp.dot(
                a_rcv[...], b_rcv[j], preferred_element_type=jnp.float32
            )
            cp = pltpu.make_async_copy(
                acc.at[j], out_hbm.at[:, pl.ds(j * nc, nc)], out_sems.at[j]
            )
            cp.start()
            out_cps.append(cp)

        for cp in out_cps:
            cp.wait()
        rdma_a.wait_send()
        for j in range(N_CHUNKS):
            rdma_bs[j].wait_send()

    return pl.pallas_call(
        body,
        out_shape=jax.ShapeDtypeStruct((m, n), jnp.float32),
        in_specs=[
            pl.BlockSpec(memory_space=pl.ANY),
            pl.BlockSpec(memory_space=pl.ANY),
        ],
        out_specs=pl.BlockSpec(memory_space=pl.ANY),
        scratch_shapes=[
            pltpu.VMEM((m, k), jnp.float32),
            pltpu.VMEM((k, n), jnp.float32),
            pltpu.VMEM((m, k), jnp.bfloat16),
            pltpu.VMEM((N_CHUNKS, k, nc), jnp.bfloat16),
            pltpu.VMEM((m, k), jnp.bfloat16),
            pltpu.VMEM((N_CHUNKS, k, nc), jnp.bfloat16),
            pltpu.VMEM((N_CHUNKS, m, nc), jnp.float32),
            pltpu.SemaphoreType.DMA((2,)),
            pltpu.SemaphoreType.DMA((N_CHUNKS,)),
            pltpu.SemaphoreType.DMA((1 + N_CHUNKS,)),
            pltpu.SemaphoreType.DMA((1 + N_CHUNKS,)),
        ],
        compiler_params=pltpu.CompilerParams(collective_id=0),
    )(A, B)
